# baseline (speedup 1.0000x reference)
import jax
import jax.numpy as jnp
from jax.experimental import pallas as pl
from jax.experimental.pallas import tpu as pltpu

TOKENS = 32768
EMBED = 1024
OUT = 64
BLK = 2048
HALF = TOKENS // 2
NSTEP = HALF // BLK
NSLOT = 3


def _proj_kernel(x_blk_ref, x_any, w_ref, b_ref, o_ref, bufb, sems):
    i = pl.program_id(0)
    w = w_ref[...]
    bias = b_ref[...]

    def copy(chunk, slot):
        return pltpu.make_async_copy(
            x_any.at[pl.ds(HALF + chunk * BLK, BLK), :],
            bufb.at[slot],
            sems.at[slot],
        )

    @pl.when(i == 0)
    def _():
        for j in range(NSLOT):
            copy(j, j).start()

    dn = (((1,), (1,)), ((), ()))
    o_ref[0] = (
        jax.lax.dot_general(x_blk_ref[...], w, dn, preferred_element_type=jnp.float32)
        + bias
    )
    slot = jax.lax.rem(i, NSLOT)
    copy(i, slot).wait()
    o_ref[1] = (
        jax.lax.dot_general(bufb[slot], w, dn, preferred_element_type=jnp.float32)
        + bias
    )

    @pl.when(i + NSLOT < NSTEP)
    def _():
        copy(i + NSLOT, slot).start()


@jax.jit
def kernel(x, W, b):
    b2 = b.reshape(1, OUT)
    out = pl.pallas_call(
        _proj_kernel,
        grid=(NSTEP,),
        in_specs=[
            pl.BlockSpec((BLK, EMBED), lambda i: (i, 0)),
            pl.BlockSpec(memory_space=pltpu.MemorySpace.HBM),
            pl.BlockSpec((OUT, EMBED), lambda i: (0, 0)),
            pl.BlockSpec((1, OUT), lambda i: (0, 0)),
        ],
        out_specs=pl.BlockSpec((2, BLK, OUT), lambda i: (0, i, 0)),
        out_shape=jax.ShapeDtypeStruct((2, HALF, OUT), jnp.float32),
        scratch_shapes=[
            pltpu.VMEM((NSLOT, BLK, EMBED), jnp.float32),
            pltpu.SemaphoreType.DMA((NSLOT,)),
        ],
        compiler_params=pltpu.CompilerParams(
            dimension_semantics=("arbitrary",),
        ),
    )(x, x, W, b2)
    return out.reshape(TOKENS, OUT)
